# baseline (speedup 1.0000x reference)
import dataclasses
import functools

import jax
import jax.numpy as jnp
from jax import lax
from jax.experimental import pallas as pl
from jax.experimental.pallas import tpu as pltpu
from jax.experimental.pallas import tpu_sc as plsc

_N = 10000
_E = 160000
_D = 256
_L = 3
_G = 64
_HALF = 128
_NC = 2
_NS = 16
_EPT = _E // _NS
_K = 80
_NCHUNK = _EPT // _K

_HIGH = jax.lax.Precision.HIGHEST
_DEF = jax.lax.Precision.DEFAULT

_ROWS_PT = _N // _NS
_PAD = 2 * _K


def _sc_agg_body(h2_hbm, src_hbm, dst_hbm, b0_hbm, b1_hbm, out_hbm,
                 src_v, srcx_v, dst_v, dstx_v, rows_v, zbuf_v, bnd_v, acc_sh):
    c = lax.axis_index("c")
    s = lax.axis_index("s")
    lane = lax.iota(jnp.int32, 16)

    pltpu.sync_copy(b0_hbm, bnd_v)
    b0 = jnp.sum(jnp.where(lane == s, bnd_v[...], 0), axis=0)
    pltpu.sync_copy(b1_hbm, bnd_v)
    b1 = jnp.sum(jnp.where(lane == s, bnd_v[...], 0), axis=0)

    @pl.loop(0, 16)
    def _(i):
        @pl.loop(0, _HALF, step=16)
        def _(j):
            zbuf_v.at[i, pl.ds(j, 16)][...] = jnp.zeros((16,), jnp.float32)

    @pl.loop(s * 16, _N + 16, step=_NS * 16)
    def _(r):
        pltpu.sync_copy(zbuf_v, acc_sh.at[pl.ds(r, 16)])

    plsc.subcore_barrier()

    start = (b0 // 8) * 8
    nch = (b1 - start + _K - 1) // _K

    @pl.loop(0, nch)
    def _(k):
        eb = start + k * _K
        pltpu.sync_copy(src_hbm.at[pl.ds(eb, _K)], src_v)
        pltpu.sync_copy(dst_hbm.at[pl.ds(eb, _K)], dst_v)

        @pl.loop(0, _K, step=16)
        def _(j):
            pos = eb + j + lane
            valid = (pos >= b0) & (pos < b1)
            dstx_v.at[pl.ds(j, 16)][...] = jnp.where(
                valid, dst_v.at[pl.ds(j, 16)][...], _N + lane)
            srcx_v.at[pl.ds(j, 16)][...] = src_v.at[pl.ds(j, 16)][...] * 2 + c

        pltpu.sync_copy(h2_hbm.at[srcx_v], rows_v)
        pltpu.sync_copy(rows_v, acc_sh.at[dstx_v], add=True)

    plsc.subcore_barrier()

    @pl.loop(s * 16, _N, step=_NS * 16)
    def _(r):
        pltpu.sync_copy(acc_sh.at[pl.ds(r, 16)], out_hbm.at[c].at[pl.ds(r, 16)])


@functools.lru_cache(maxsize=1)
def _get_sc_agg():
    mesh = plsc.VectorSubcoreMesh(
        core_axis_name="c", subcore_axis_name="s",
        num_cores=_NC, num_subcores=_NS)
    cp = pltpu.CompilerParams()
    if "needs_layout_passes" in pltpu.CompilerParams.__dataclass_fields__:
        cp = dataclasses.replace(cp, needs_layout_passes=False)
    return pl.kernel(
        _sc_agg_body,
        out_type=jax.ShapeDtypeStruct((_NC, _N, _HALF), jnp.float32),
        mesh=mesh,
        scratch_types=[
            pltpu.VMEM((_K,), jnp.int32),
            pltpu.VMEM((_K,), jnp.int32),
            pltpu.VMEM((_K,), jnp.int32),
            pltpu.VMEM((_K,), jnp.int32),
            pltpu.VMEM((_K, _HALF), jnp.float32),
            pltpu.VMEM((16, _HALF), jnp.float32),
            pltpu.VMEM((16,), jnp.int32),
            pltpu.VMEM_SHARED((_N + 16, _HALF), jnp.float32),
        ],
        compiler_params=cp,
    )


def _enc_body(x_ref, w_ref, b_ref, o_ref):
    o_ref[...] = x_ref[...] * w_ref[...] + b_ref[...]


def _bn(z, g, b):
    m = jnp.mean(z, axis=0, keepdims=True)
    v = jnp.mean((z - m) ** 2, axis=0, keepdims=True)
    return g * (z - m) / jnp.sqrt(v + 1e-5) + b


def _layer_body(h_ref, a0_ref, a1_ref, ep_ref, w1_ref, b1_ref, g1_ref, be1_ref,
                w2_ref, b2_ref, go_ref, bo_ref, o_ref):
    agg = jnp.concatenate([a0_ref[...], a1_ref[...]], axis=1)
    z = ep_ref[...] * h_ref[...] + agg
    z = jnp.dot(z, w1_ref[...], precision=_DEF,
                preferred_element_type=jnp.float32) + b1_ref[...]
    z = jnp.maximum(_bn(z, g1_ref[...], be1_ref[...]), 0.0)
    z = jnp.dot(z, w2_ref[...], precision=_DEF,
                preferred_element_type=jnp.float32) + b2_ref[...]
    o_ref[...] = jnp.maximum(_bn(z, go_ref[...], bo_ref[...]), 0.0)


def _final_body(h_ref, bt_ref, cw_ref, cb_ref, o_ref):
    ids = lax.broadcasted_iota(jnp.int32, (_G, 1), 0)
    oh = (ids == bt_ref[...]).astype(jnp.float32)
    sums = jnp.dot(oh, h_ref[...], precision=_HIGH,
                   preferred_element_type=jnp.float32)
    cnts = jnp.sum(oh, axis=1, keepdims=True)
    pooled = sums / jnp.maximum(cnts, 1.0)
    o_ref[...] = jnp.dot(pooled, cw_ref[...], precision=_DEF,
                         preferred_element_type=jnp.float32) + cb_ref[...]


_enc_call = pl.pallas_call(
    _enc_body, out_shape=jax.ShapeDtypeStruct((_N, _D), jnp.float32))

_layer_call = pl.pallas_call(
    _layer_body, out_shape=jax.ShapeDtypeStruct((_N, _D), jnp.float32),
    compiler_params=pltpu.CompilerParams(vmem_limit_bytes=64 * 1024 * 1024))

_final_call = pl.pallas_call(
    _final_body, out_shape=jax.ShapeDtypeStruct((_G, 2), jnp.float32))


def kernel(x, edge_index, batch, enc_W, enc_b, W1, b1, g1, be1, W2, b2, eps,
           g_out, b_out, cls_W, cls_b):
    src = edge_index[0]
    dst = edge_index[1]

    order = jnp.argsort(dst, stable=True)
    srcs = src[order]
    dsts = dst[order]
    bnds = jnp.searchsorted(
        dsts, jnp.arange(0, _N + _ROWS_PT, _ROWS_PT), side="left"
    ).astype(jnp.int32)
    b0s = bnds[0:_NS]
    b1s = bnds[1:_NS + 1]
    pad_src = (jnp.arange(_PAD, dtype=jnp.int32) * 97) % _N
    srcs = jnp.concatenate([srcs, pad_src])
    dsts = jnp.concatenate([dsts, jnp.zeros((_PAD,), jnp.int32)])

    h = _enc_call(x, enc_W, enc_b.reshape(1, _D))
    for l in range(_L):
        aggp = _get_sc_agg()(h.reshape(2 * _N, _HALF), srcs, dsts, b0s, b1s)
        h = _layer_call(
            h, aggp[0], aggp[1],
            (1.0 + eps[l]).reshape(1, 1),
            W1[l], b1[l].reshape(1, _D), g1[l].reshape(1, _D),
            be1[l].reshape(1, _D),
            W2[l], b2[l].reshape(1, _D),
            g_out[l].reshape(1, _D), b_out[l].reshape(1, _D),
        )
    return _final_call(h, batch.reshape(1, _N), cls_W, cls_b.reshape(1, 2))


# speedup vs baseline: 1.3580x; 1.3580x over previous
import dataclasses
import functools

import jax
import jax.numpy as jnp
from jax import lax
from jax.experimental import pallas as pl
from jax.experimental.pallas import tpu as pltpu
from jax.experimental.pallas import tpu_sc as plsc

_N = 10000
_E = 160000
_D = 256
_L = 3
_G = 64
_HALF = 128
_NC = 2
_NS = 16
_EPT = _E // _NS
_K = 80
_NCHUNK = _EPT // _K

_HIGH = jax.lax.Precision.HIGHEST
_DEF = jax.lax.Precision.DEFAULT

_ROWS_PT = _N // _NS
_PAD = 2 * _K


def _sc_agg_body(h2_hbm, src_hbm, dst_hbm, b0_hbm, b1_hbm, out_hbm,
                 src_v0, srcx_v0, dst_v0, dstx_v0, rows_v0,
                 src_v1, srcx_v1, dst_v1, dstx_v1, rows_v1,
                 sem0, sem1, zbuf_v, bnd_v, acc_sh):
    c = lax.axis_index("c")
    s = lax.axis_index("s")
    lane = lax.iota(jnp.int32, 16)
    bufs = ((src_v0, srcx_v0, dst_v0, dstx_v0, rows_v0, sem0),
            (src_v1, srcx_v1, dst_v1, dstx_v1, rows_v1, sem1))

    pltpu.sync_copy(b0_hbm, bnd_v)
    b0 = jnp.sum(jnp.where(lane == s, bnd_v[...], 0), axis=0)
    pltpu.sync_copy(b1_hbm, bnd_v)
    b1 = jnp.sum(jnp.where(lane == s, bnd_v[...], 0), axis=0)

    @pl.loop(0, 16)
    def _(i):
        @pl.loop(0, _HALF, step=16)
        def _(j):
            zbuf_v.at[i, pl.ds(j, 16)][...] = jnp.zeros((16,), jnp.float32)

    @pl.loop(s * 16, _N + 16, step=_NS * 16)
    def _(r):
        pltpu.sync_copy(zbuf_v, acc_sh.at[pl.ds(r, 16)])

    plsc.subcore_barrier()

    start = (b0 // 8) * 8
    nch = (b1 - start + _K - 1) // _K

    def prefetch(k, buf):
        src_v, srcx_v, dst_v, dstx_v, rows_v, sem = buf
        eb = start + k * _K
        pltpu.sync_copy(src_hbm.at[pl.ds(eb, _K)], src_v)
        pltpu.sync_copy(dst_hbm.at[pl.ds(eb, _K)], dst_v)

        @pl.loop(0, _K, step=16)
        def _(j):
            pos = eb + j + lane
            valid = (pos >= b0) & (pos < b1)
            dstx_v.at[pl.ds(j, 16)][...] = jnp.where(
                valid, dst_v.at[pl.ds(j, 16)][...], _N + lane)
            srcx_v.at[pl.ds(j, 16)][...] = src_v.at[pl.ds(j, 16)][...] * 2 + c

        pltpu.async_copy(h2_hbm.at[srcx_v], rows_v, sem)

    def finish(buf):
        src_v, srcx_v, dst_v, dstx_v, rows_v, sem = buf
        pltpu.make_async_copy(h2_hbm.at[srcx_v], rows_v, sem).wait()
        pltpu.sync_copy(rows_v, acc_sh.at[dstx_v], add=True)

    @pl.when(nch > 0)
    def _():
        prefetch(0, bufs[0])

    @pl.loop(0, nch)
    def _(k):
        even = k % 2 == 0

        @pl.when((k + 1 < nch) & even)
        def _():
            prefetch(k + 1, bufs[1])

        @pl.when((k + 1 < nch) & jnp.logical_not(even))
        def _():
            prefetch(k + 1, bufs[0])

        @pl.when(even)
        def _():
            finish(bufs[0])

        @pl.when(jnp.logical_not(even))
        def _():
            finish(bufs[1])

    plsc.subcore_barrier()

    @pl.loop(s * 16, _N, step=_NS * 16)
    def _(r):
        pltpu.sync_copy(acc_sh.at[pl.ds(r, 16)], out_hbm.at[c].at[pl.ds(r, 16)])


@functools.lru_cache(maxsize=1)
def _get_sc_agg():
    mesh = plsc.VectorSubcoreMesh(
        core_axis_name="c", subcore_axis_name="s",
        num_cores=_NC, num_subcores=_NS)
    cp = pltpu.CompilerParams()
    if "needs_layout_passes" in pltpu.CompilerParams.__dataclass_fields__:
        cp = dataclasses.replace(cp, needs_layout_passes=False)
    return pl.kernel(
        _sc_agg_body,
        out_type=jax.ShapeDtypeStruct((_NC, _N, _HALF), jnp.float32),
        mesh=mesh,
        scratch_types=(
            2 * [
                pltpu.VMEM((_K,), jnp.int32),
                pltpu.VMEM((_K,), jnp.int32),
                pltpu.VMEM((_K,), jnp.int32),
                pltpu.VMEM((_K,), jnp.int32),
                pltpu.VMEM((_K, _HALF), jnp.float32),
            ]
            + [
                pltpu.SemaphoreType.DMA,
                pltpu.SemaphoreType.DMA,
                pltpu.VMEM((16, _HALF), jnp.float32),
                pltpu.VMEM((16,), jnp.int32),
                pltpu.VMEM_SHARED((_N + 16, _HALF), jnp.float32),
            ]
        ),
        compiler_params=cp,
    )


def _enc_body(x_ref, w_ref, b_ref, o_ref):
    o_ref[...] = x_ref[...] * w_ref[...] + b_ref[...]


def _bn(z, g, b):
    m = jnp.mean(z, axis=0, keepdims=True)
    v = jnp.mean((z - m) ** 2, axis=0, keepdims=True)
    return g * (z - m) / jnp.sqrt(v + 1e-5) + b


def _layer_body(h_ref, a0_ref, a1_ref, ep_ref, w1_ref, b1_ref, g1_ref, be1_ref,
                w2_ref, b2_ref, go_ref, bo_ref, o_ref):
    agg = jnp.concatenate([a0_ref[...], a1_ref[...]], axis=1)
    z = ep_ref[...] * h_ref[...] + agg
    z = jnp.dot(z, w1_ref[...], precision=_DEF,
                preferred_element_type=jnp.float32) + b1_ref[...]
    z = jnp.maximum(_bn(z, g1_ref[...], be1_ref[...]), 0.0)
    z = jnp.dot(z, w2_ref[...], precision=_DEF,
                preferred_element_type=jnp.float32) + b2_ref[...]
    o_ref[...] = jnp.maximum(_bn(z, go_ref[...], bo_ref[...]), 0.0)


def _final_body(h_ref, bt_ref, cw_ref, cb_ref, o_ref):
    ids = lax.broadcasted_iota(jnp.int32, (_G, 1), 0)
    oh = (ids == bt_ref[...]).astype(jnp.float32)
    sums = jnp.dot(oh, h_ref[...], precision=_HIGH,
                   preferred_element_type=jnp.float32)
    cnts = jnp.sum(oh, axis=1, keepdims=True)
    pooled = sums / jnp.maximum(cnts, 1.0)
    o_ref[...] = jnp.dot(pooled, cw_ref[...], precision=_DEF,
                         preferred_element_type=jnp.float32) + cb_ref[...]


_enc_call = pl.pallas_call(
    _enc_body, out_shape=jax.ShapeDtypeStruct((_N, _D), jnp.float32))

_layer_call = pl.pallas_call(
    _layer_body, out_shape=jax.ShapeDtypeStruct((_N, _D), jnp.float32),
    compiler_params=pltpu.CompilerParams(vmem_limit_bytes=64 * 1024 * 1024))

_final_call = pl.pallas_call(
    _final_body, out_shape=jax.ShapeDtypeStruct((_G, 2), jnp.float32))


def kernel(x, edge_index, batch, enc_W, enc_b, W1, b1, g1, be1, W2, b2, eps,
           g_out, b_out, cls_W, cls_b):
    src = edge_index[0]
    dst = edge_index[1]

    order = jnp.argsort(dst, stable=True)
    srcs = src[order]
    dsts = dst[order]
    bnds = jnp.searchsorted(
        dsts, jnp.arange(0, _N + _ROWS_PT, _ROWS_PT), side="left"
    ).astype(jnp.int32)
    b0s = bnds[0:_NS]
    b1s = bnds[1:_NS + 1]
    pad_src = (jnp.arange(_PAD, dtype=jnp.int32) * 97) % _N
    srcs = jnp.concatenate([srcs, pad_src])
    dsts = jnp.concatenate([dsts, jnp.zeros((_PAD,), jnp.int32)])

    h = _enc_call(x, enc_W, enc_b.reshape(1, _D))
    for l in range(_L):
        aggp = _get_sc_agg()(h.reshape(2 * _N, _HALF), srcs, dsts, b0s, b1s)
        h = _layer_call(
            h, aggp[0], aggp[1],
            (1.0 + eps[l]).reshape(1, 1),
            W1[l], b1[l].reshape(1, _D), g1[l].reshape(1, _D),
            be1[l].reshape(1, _D),
            W2[l], b2[l].reshape(1, _D),
            g_out[l].reshape(1, _D), b_out[l].reshape(1, _D),
        )
    return _final_call(h, batch.reshape(1, _N), cls_W, cls_b.reshape(1, 2))


# speedup vs baseline: 1.5862x; 1.1681x over previous
import dataclasses
import functools

import jax
import jax.numpy as jnp
from jax import lax
from jax.experimental import pallas as pl
from jax.experimental.pallas import tpu as pltpu
from jax.experimental.pallas import tpu_sc as plsc

_N = 10000
_E = 160000
_D = 256
_L = 3
_G = 64
_HALF = 128
_NC = 2
_NS = 16
_EPT = _E // _NS
_K = 80
_NCHUNK = _EPT // _K

_HIGH = jax.lax.Precision.HIGHEST
_DEF = jax.lax.Precision.DEFAULT

_ROWS_PT = _N // _NS
_IDXB = 10
_PAD = (_IDXB + 2) * _K


def _sc_agg_body(h2_hbm, src_hbm, dst_hbm, b0_hbm, b1_hbm, out_hbm,
                 srcx_v0, dstx_v0, rows_v0,
                 srcx_v1, dstx_v1, rows_v1,
                 sem0, sem1, src_blk, dst_blk, zbuf_v, bnd_v, acc_sh):
    c = lax.axis_index("c")
    s = lax.axis_index("s")
    lane = lax.iota(jnp.int32, 16)
    bufs = ((srcx_v0, dstx_v0, rows_v0, sem0),
            (srcx_v1, dstx_v1, rows_v1, sem1))

    pltpu.sync_copy(b0_hbm, bnd_v)
    b0 = jnp.sum(jnp.where(lane == s, bnd_v[...], 0), axis=0)
    pltpu.sync_copy(b1_hbm, bnd_v)
    b1 = jnp.sum(jnp.where(lane == s, bnd_v[...], 0), axis=0)

    @pl.loop(0, 16)
    def _(i):
        @pl.loop(0, _HALF, step=16)
        def _(j):
            zbuf_v.at[i, pl.ds(j, 16)][...] = jnp.zeros((16,), jnp.float32)

    @pl.loop(s * 16, _N + 16, step=_NS * 16)
    def _(r):
        pltpu.sync_copy(zbuf_v, acc_sh.at[pl.ds(r, 16)])

    plsc.subcore_barrier()

    start = (b0 // 8) * 8
    nch = (b1 - start + _K - 1) // _K

    def prefetch(k, buf):
        srcx_v, dstx_v, rows_v, sem = buf
        eb = start + k * _K
        off = (k % _IDXB) * _K

        @pl.when(off == 0)
        def _():
            pltpu.sync_copy(src_hbm.at[pl.ds(eb, _IDXB * _K)], src_blk)
            pltpu.sync_copy(dst_hbm.at[pl.ds(eb, _IDXB * _K)], dst_blk)

        @pl.loop(0, _K, step=16)
        def _(j):
            pos = eb + j + lane
            valid = (pos >= b0) & (pos < b1)
            dstx_v.at[pl.ds(j, 16)][...] = jnp.where(
                valid, dst_blk.at[pl.ds(off + j, 16)][...], _N + lane)
            srcx_v.at[pl.ds(j, 16)][...] = src_blk.at[pl.ds(off + j, 16)][...] * 2 + c

        pltpu.async_copy(h2_hbm.at[srcx_v], rows_v, sem)

    def finish(buf):
        srcx_v, dstx_v, rows_v, sem = buf
        pltpu.make_async_copy(h2_hbm.at[srcx_v], rows_v, sem).wait()
        pltpu.sync_copy(rows_v, acc_sh.at[dstx_v], add=True)

    @pl.when(nch > 0)
    def _():
        prefetch(0, bufs[0])

    @pl.loop(0, nch)
    def _(k):
        even = k % 2 == 0

        @pl.when((k + 1 < nch) & even)
        def _():
            prefetch(k + 1, bufs[1])

        @pl.when((k + 1 < nch) & jnp.logical_not(even))
        def _():
            prefetch(k + 1, bufs[0])

        @pl.when(even)
        def _():
            finish(bufs[0])

        @pl.when(jnp.logical_not(even))
        def _():
            finish(bufs[1])

    plsc.subcore_barrier()

    @pl.loop(s * 16, _N, step=_NS * 16)
    def _(r):
        pltpu.sync_copy(acc_sh.at[pl.ds(r, 16)], out_hbm.at[c].at[pl.ds(r, 16)])


@functools.lru_cache(maxsize=1)
def _get_sc_agg():
    mesh = plsc.VectorSubcoreMesh(
        core_axis_name="c", subcore_axis_name="s",
        num_cores=_NC, num_subcores=_NS)
    cp = pltpu.CompilerParams()
    if "needs_layout_passes" in pltpu.CompilerParams.__dataclass_fields__:
        cp = dataclasses.replace(cp, needs_layout_passes=False)
    return pl.kernel(
        _sc_agg_body,
        out_type=jax.ShapeDtypeStruct((_NC, _N, _HALF), jnp.float32),
        mesh=mesh,
        scratch_types=(
            2 * [
                pltpu.VMEM((_K,), jnp.int32),
                pltpu.VMEM((_K,), jnp.int32),
                pltpu.VMEM((_K, _HALF), jnp.float32),
            ]
            + [
                pltpu.SemaphoreType.DMA,
                pltpu.SemaphoreType.DMA,
                pltpu.VMEM((_IDXB * _K,), jnp.int32),
                pltpu.VMEM((_IDXB * _K,), jnp.int32),
                pltpu.VMEM((16, _HALF), jnp.float32),
                pltpu.VMEM((16,), jnp.int32),
                pltpu.VMEM_SHARED((_N + 16, _HALF), jnp.float32),
            ]
        ),
        compiler_params=cp,
    )


def _enc_body(x_ref, w_ref, b_ref, o_ref):
    o_ref[...] = x_ref[...] * w_ref[...] + b_ref[...]


def _bn(z, g, b):
    m = jnp.mean(z, axis=0, keepdims=True)
    v = jnp.mean((z - m) ** 2, axis=0, keepdims=True)
    return g * (z - m) / jnp.sqrt(v + 1e-5) + b


def _layer_body(h_ref, a0_ref, a1_ref, ep_ref, w1_ref, b1_ref, g1_ref, be1_ref,
                w2_ref, b2_ref, go_ref, bo_ref, o_ref):
    agg = jnp.concatenate([a0_ref[...], a1_ref[...]], axis=1)
    z = ep_ref[...] * h_ref[...] + agg
    z = jnp.dot(z, w1_ref[...], precision=_DEF,
                preferred_element_type=jnp.float32) + b1_ref[...]
    z = jnp.maximum(_bn(z, g1_ref[...], be1_ref[...]), 0.0)
    z = jnp.dot(z, w2_ref[...], precision=_DEF,
                preferred_element_type=jnp.float32) + b2_ref[...]
    o_ref[...] = jnp.maximum(_bn(z, go_ref[...], bo_ref[...]), 0.0)


def _final_body(h_ref, bt_ref, cw_ref, cb_ref, o_ref):
    ids = lax.broadcasted_iota(jnp.int32, (_G, 1), 0)
    oh = (ids == bt_ref[...]).astype(jnp.float32)
    sums = jnp.dot(oh, h_ref[...], precision=_HIGH,
                   preferred_element_type=jnp.float32)
    cnts = jnp.sum(oh, axis=1, keepdims=True)
    pooled = sums / jnp.maximum(cnts, 1.0)
    o_ref[...] = jnp.dot(pooled, cw_ref[...], precision=_DEF,
                         preferred_element_type=jnp.float32) + cb_ref[...]


_enc_call = pl.pallas_call(
    _enc_body, out_shape=jax.ShapeDtypeStruct((_N, _D), jnp.float32))

_layer_call = pl.pallas_call(
    _layer_body, out_shape=jax.ShapeDtypeStruct((_N, _D), jnp.float32),
    compiler_params=pltpu.CompilerParams(vmem_limit_bytes=64 * 1024 * 1024))

_final_call = pl.pallas_call(
    _final_body, out_shape=jax.ShapeDtypeStruct((_G, 2), jnp.float32))


def kernel(x, edge_index, batch, enc_W, enc_b, W1, b1, g1, be1, W2, b2, eps,
           g_out, b_out, cls_W, cls_b):
    src = edge_index[0]
    dst = edge_index[1]

    order = jnp.argsort(dst, stable=True)
    srcs = src[order]
    dsts = dst[order]
    bnds = jnp.searchsorted(
        dsts, jnp.arange(0, _N + _ROWS_PT, _ROWS_PT), side="left"
    ).astype(jnp.int32)
    b0s = bnds[0:_NS]
    b1s = bnds[1:_NS + 1]
    pad_src = (jnp.arange(_PAD, dtype=jnp.int32) * 97) % _N
    srcs = jnp.concatenate([srcs, pad_src])
    dsts = jnp.concatenate([dsts, jnp.zeros((_PAD,), jnp.int32)])

    h = _enc_call(x, enc_W, enc_b.reshape(1, _D))
    for l in range(_L):
        aggp = _get_sc_agg()(h.reshape(2 * _N, _HALF), srcs, dsts, b0s, b1s)
        h = _layer_call(
            h, aggp[0], aggp[1],
            (1.0 + eps[l]).reshape(1, 1),
            W1[l], b1[l].reshape(1, _D), g1[l].reshape(1, _D),
            be1[l].reshape(1, _D),
            W2[l], b2[l].reshape(1, _D),
            g_out[l].reshape(1, _D), b_out[l].reshape(1, _D),
        )
    return _final_call(h, batch.reshape(1, _N), cls_W, cls_b.reshape(1, 2))


# speedup vs baseline: 1.7564x; 1.1073x over previous
import dataclasses
import functools

import jax
import jax.numpy as jnp
from jax import lax
from jax.experimental import pallas as pl
from jax.experimental.pallas import tpu as pltpu
from jax.experimental.pallas import tpu_sc as plsc

_N = 10000
_E = 160000
_D = 256
_L = 3
_G = 64
_HALF = 128
_NC = 2
_NS = 16
_EPT = _E // _NS
_K = 80
_NCHUNK = _EPT // _K

_HIGH = jax.lax.Precision.HIGHEST
_DEF = jax.lax.Precision.DEFAULT

_ROWS_PT = _N // _NS
_IDXB = 10
_PAD = (_IDXB + 2) * _K


def _sc_agg_body(h2_hbm, src_hbm, dst_hbm, b0_hbm, b1_hbm, out_hbm,
                 srcx_v0, dstx_v0, rows_v0,
                 srcx_v1, dstx_v1, rows_v1,
                 sem0, sem1, src_blk, dst_blk, zbuf_v, bnd_v, acc_sh):
    c = lax.axis_index("c")
    s = lax.axis_index("s")
    lane = lax.iota(jnp.int32, 16)
    bufs = ((srcx_v0, dstx_v0, rows_v0, sem0),
            (srcx_v1, dstx_v1, rows_v1, sem1))

    pltpu.sync_copy(b0_hbm, bnd_v)
    b0 = jnp.sum(jnp.where(lane == s, bnd_v[...], 0), axis=0)
    pltpu.sync_copy(b1_hbm, bnd_v)
    b1 = jnp.sum(jnp.where(lane == s, bnd_v[...], 0), axis=0)

    @pl.loop(0, 16)
    def _(i):
        @pl.loop(0, _HALF, step=16)
        def _(j):
            zbuf_v.at[i, pl.ds(j, 16)][...] = jnp.zeros((16,), jnp.float32)

    @pl.loop(s * 16, _N + 16, step=_NS * 16)
    def _(r):
        pltpu.sync_copy(zbuf_v, acc_sh.at[pl.ds(r, 16)])

    plsc.subcore_barrier()

    start = (b0 // 8) * 8
    nch = (b1 - start + _K - 1) // _K

    def prefetch(k, buf):
        srcx_v, dstx_v, rows_v, sem = buf
        eb = start + k * _K
        off = (k % _IDXB) * _K

        @pl.when(off == 0)
        def _():
            pltpu.sync_copy(src_hbm.at[pl.ds(eb, _IDXB * _K)], src_blk)
            pltpu.sync_copy(dst_hbm.at[pl.ds(eb, _IDXB * _K)], dst_blk)

        @pl.loop(0, _K, step=16)
        def _(j):
            pos = eb + j + lane
            valid = (pos >= b0) & (pos < b1)
            dstx_v.at[pl.ds(j, 16)][...] = jnp.where(
                valid, dst_blk.at[pl.ds(off + j, 16)][...], _N + lane)
            srcx_v.at[pl.ds(j, 16)][...] = src_blk.at[pl.ds(off + j, 16)][...] * 2 + c

        pltpu.async_copy(h2_hbm.at[srcx_v], rows_v, sem)

    def finish(buf):
        srcx_v, dstx_v, rows_v, sem = buf
        pltpu.make_async_copy(h2_hbm.at[srcx_v], rows_v, sem).wait()
        pltpu.sync_copy(rows_v, acc_sh.at[dstx_v], add=True)

    @pl.when(nch > 0)
    def _():
        prefetch(0, bufs[0])

    @pl.loop(0, nch)
    def _(k):
        even = k % 2 == 0

        @pl.when((k + 1 < nch) & even)
        def _():
            prefetch(k + 1, bufs[1])

        @pl.when((k + 1 < nch) & jnp.logical_not(even))
        def _():
            prefetch(k + 1, bufs[0])

        @pl.when(even)
        def _():
            finish(bufs[0])

        @pl.when(jnp.logical_not(even))
        def _():
            finish(bufs[1])

    plsc.subcore_barrier()

    @pl.loop(s * 16, _N, step=_NS * 16)
    def _(r):
        pltpu.sync_copy(acc_sh.at[pl.ds(r, 16)], out_hbm.at[c].at[pl.ds(r, 16)])


@functools.lru_cache(maxsize=1)
def _get_sc_agg():
    mesh = plsc.VectorSubcoreMesh(
        core_axis_name="c", subcore_axis_name="s",
        num_cores=_NC, num_subcores=_NS)
    cp = pltpu.CompilerParams()
    if "needs_layout_passes" in pltpu.CompilerParams.__dataclass_fields__:
        cp = dataclasses.replace(cp, needs_layout_passes=False)
    return pl.kernel(
        _sc_agg_body,
        out_type=jax.ShapeDtypeStruct((_NC, _N, _HALF), jnp.float32),
        mesh=mesh,
        scratch_types=(
            2 * [
                pltpu.VMEM((_K,), jnp.int32),
                pltpu.VMEM((_K,), jnp.int32),
                pltpu.VMEM((_K, _HALF), jnp.float32),
            ]
            + [
                pltpu.SemaphoreType.DMA,
                pltpu.SemaphoreType.DMA,
                pltpu.VMEM((_IDXB * _K,), jnp.int32),
                pltpu.VMEM((_IDXB * _K,), jnp.int32),
                pltpu.VMEM((16, _HALF), jnp.float32),
                pltpu.VMEM((16,), jnp.int32),
                pltpu.VMEM_SHARED((_N + 16, _HALF), jnp.float32),
            ]
        ),
        compiler_params=cp,
    )


def _enc_body(x_ref, w_ref, b_ref, o_ref):
    o_ref[...] = x_ref[...] * w_ref[...] + b_ref[...]


def _bn(z, g, b):
    m = jnp.mean(z, axis=0, keepdims=True)
    v = jnp.mean((z - m) ** 2, axis=0, keepdims=True)
    return g * (z - m) / jnp.sqrt(v + 1e-5) + b


def _layer_body(h_ref, a0_ref, a1_ref, ep_ref, w1_ref, b1_ref, g1_ref, be1_ref,
                w2_ref, b2_ref, go_ref, bo_ref, o_ref):
    agg = jnp.concatenate([a0_ref[...], a1_ref[...]], axis=1)
    z = ep_ref[...] * h_ref[...] + agg
    z = jnp.dot(z, w1_ref[...], precision=_DEF,
                preferred_element_type=jnp.float32) + b1_ref[...]
    z = jnp.maximum(_bn(z, g1_ref[...], be1_ref[...]), 0.0)
    z = jnp.dot(z, w2_ref[...], precision=_DEF,
                preferred_element_type=jnp.float32) + b2_ref[...]
    o_ref[...] = jnp.maximum(_bn(z, go_ref[...], bo_ref[...]), 0.0)


def _final_body(h_ref, bt_ref, cw_ref, cb_ref, o_ref):
    ids = lax.broadcasted_iota(jnp.int32, (_G, 1), 0)
    oh = (ids == bt_ref[...]).astype(jnp.float32)
    sums = jnp.dot(oh, h_ref[...], precision=_HIGH,
                   preferred_element_type=jnp.float32)
    cnts = jnp.sum(oh, axis=1, keepdims=True)
    pooled = sums / jnp.maximum(cnts, 1.0)
    o_ref[...] = jnp.dot(pooled, cw_ref[...], precision=_DEF,
                         preferred_element_type=jnp.float32) + cb_ref[...]


_enc_call = pl.pallas_call(
    _enc_body, out_shape=jax.ShapeDtypeStruct((_N, _D), jnp.float32))

_layer_call = pl.pallas_call(
    _layer_body, out_shape=jax.ShapeDtypeStruct((_N, _D), jnp.float32),
    compiler_params=pltpu.CompilerParams(vmem_limit_bytes=64 * 1024 * 1024))

_final_call = pl.pallas_call(
    _final_body, out_shape=jax.ShapeDtypeStruct((_G, 2), jnp.float32))


def kernel(x, edge_index, batch, enc_W, enc_b, W1, b1, g1, be1, W2, b2, eps,
           g_out, b_out, cls_W, cls_b):
    src = edge_index[0]
    dst = edge_index[1]

    b = dst // _ROWS_PT
    oh = (b[:, None] == jnp.arange(_NS, dtype=jnp.int32)[None, :]).astype(jnp.int32)
    csum = jnp.cumsum(oh, axis=0)
    counts = csum[-1]
    offs = jnp.concatenate([jnp.zeros((1,), jnp.int32),
                            jnp.cumsum(counts)[:-1].astype(jnp.int32)])
    newpos = offs[b] + jnp.take_along_axis(csum, b[:, None], axis=1)[:, 0] - 1
    inv = jnp.zeros((_E,), jnp.int32).at[newpos].add(jnp.arange(_E, dtype=jnp.int32))
    srcs = src[inv]
    dsts = dst[inv]
    b0s = offs
    b1s = offs + counts.astype(jnp.int32)
    pad_src = (jnp.arange(_PAD, dtype=jnp.int32) * 97) % _N
    srcs = jnp.concatenate([srcs, pad_src])
    dsts = jnp.concatenate([dsts, jnp.zeros((_PAD,), jnp.int32)])

    h = _enc_call(x, enc_W, enc_b.reshape(1, _D))
    for l in range(_L):
        aggp = _get_sc_agg()(h.reshape(2 * _N, _HALF), srcs, dsts, b0s, b1s)
        h = _layer_call(
            h, aggp[0], aggp[1],
            (1.0 + eps[l]).reshape(1, 1),
            W1[l], b1[l].reshape(1, _D), g1[l].reshape(1, _D),
            be1[l].reshape(1, _D),
            W2[l], b2[l].reshape(1, _D),
            g_out[l].reshape(1, _D), b_out[l].reshape(1, _D),
        )
    return _final_call(h, batch.reshape(1, _N), cls_W, cls_b.reshape(1, 2))


# speedup vs baseline: 1.8979x; 1.0806x over previous
import dataclasses
import functools

import jax
import jax.numpy as jnp
from jax import lax
from jax.experimental import pallas as pl
from jax.experimental.pallas import tpu as pltpu
from jax.experimental.pallas import tpu_sc as plsc

_N = 10000
_E = 160000
_D = 256
_L = 3
_G = 64
_HALF = 128
_NC = 2
_NS = 16
_EPT = _E // _NS
_K = 128

_HIGH = jax.lax.Precision.HIGHEST
_DEF = jax.lax.Precision.DEFAULT

_ROWS_PT = _N // _NS
_IDXB = 10
_PAD = (_IDXB + 2) * _K


def _sc_agg_body(h2_hbm, src_hbm, dst_hbm, b0_hbm, b1_hbm, out_hbm,
                 srcx_v0, dstx_v0, rows_v0,
                 srcx_v1, dstx_v1, rows_v1,
                 sem0, sem1, src_blk, dst_blk, zbuf_v, bnd_v, acc_sh):
    c = lax.axis_index("c")
    s = lax.axis_index("s")
    lane = lax.iota(jnp.int32, 16)
    bufs = ((srcx_v0, dstx_v0, rows_v0, sem0),
            (srcx_v1, dstx_v1, rows_v1, sem1))

    pltpu.sync_copy(b0_hbm, bnd_v)
    b0 = jnp.sum(jnp.where(lane == s, bnd_v[...], 0), axis=0)
    pltpu.sync_copy(b1_hbm, bnd_v)
    b1 = jnp.sum(jnp.where(lane == s, bnd_v[...], 0), axis=0)

    @pl.loop(0, 16)
    def _(i):
        @pl.loop(0, _HALF, step=16)
        def _(j):
            zbuf_v.at[i, pl.ds(j, 16)][...] = jnp.zeros((16,), jnp.float32)

    @pl.loop(s * 16, _N + 16, step=_NS * 16)
    def _(r):
        pltpu.sync_copy(zbuf_v, acc_sh.at[pl.ds(r, 16)])

    plsc.subcore_barrier()

    start = (b0 // 8) * 8
    nch = (b1 - start + _K - 1) // _K

    def prefetch(k, buf):
        srcx_v, dstx_v, rows_v, sem = buf
        eb = start + k * _K
        off = (k % _IDXB) * _K

        @pl.when(off == 0)
        def _():
            pltpu.sync_copy(src_hbm.at[pl.ds(eb, _IDXB * _K)], src_blk)
            pltpu.sync_copy(dst_hbm.at[pl.ds(eb, _IDXB * _K)], dst_blk)

        @pl.loop(0, _K, step=16)
        def _(j):
            pos = eb + j + lane
            valid = (pos >= b0) & (pos < b1)
            dstx_v.at[pl.ds(j, 16)][...] = jnp.where(
                valid, dst_blk.at[pl.ds(off + j, 16)][...], _N + lane)
            srcx_v.at[pl.ds(j, 16)][...] = src_blk.at[pl.ds(off + j, 16)][...] * 2 + c

        pltpu.async_copy(h2_hbm.at[srcx_v], rows_v, sem)

    def finish(buf):
        srcx_v, dstx_v, rows_v, sem = buf
        pltpu.make_async_copy(h2_hbm.at[srcx_v], rows_v, sem).wait()
        pltpu.sync_copy(rows_v, acc_sh.at[dstx_v], add=True)

    @pl.when(nch > 0)
    def _():
        prefetch(0, bufs[0])

    @pl.loop(0, nch)
    def _(k):
        even = k % 2 == 0

        @pl.when((k + 1 < nch) & even)
        def _():
            prefetch(k + 1, bufs[1])

        @pl.when((k + 1 < nch) & jnp.logical_not(even))
        def _():
            prefetch(k + 1, bufs[0])

        @pl.when(even)
        def _():
            finish(bufs[0])

        @pl.when(jnp.logical_not(even))
        def _():
            finish(bufs[1])

    plsc.subcore_barrier()

    @pl.loop(s * 16, _N, step=_NS * 16)
    def _(r):
        pltpu.sync_copy(acc_sh.at[pl.ds(r, 16)], out_hbm.at[c].at[pl.ds(r, 16)])


@functools.lru_cache(maxsize=1)
def _get_sc_agg():
    mesh = plsc.VectorSubcoreMesh(
        core_axis_name="c", subcore_axis_name="s",
        num_cores=_NC, num_subcores=_NS)
    cp = pltpu.CompilerParams()
    if "needs_layout_passes" in pltpu.CompilerParams.__dataclass_fields__:
        cp = dataclasses.replace(cp, needs_layout_passes=False)
    return pl.kernel(
        _sc_agg_body,
        out_type=jax.ShapeDtypeStruct((_NC, _N, _HALF), jnp.float32),
        mesh=mesh,
        scratch_types=(
            2 * [
                pltpu.VMEM((_K,), jnp.int32),
                pltpu.VMEM((_K,), jnp.int32),
                pltpu.VMEM((_K, _HALF), jnp.float32),
            ]
            + [
                pltpu.SemaphoreType.DMA,
                pltpu.SemaphoreType.DMA,
                pltpu.VMEM((_IDXB * _K,), jnp.int32),
                pltpu.VMEM((_IDXB * _K,), jnp.int32),
                pltpu.VMEM((16, _HALF), jnp.float32),
                pltpu.VMEM((16,), jnp.int32),
                pltpu.VMEM_SHARED((_N + 16, _HALF), jnp.float32),
            ]
        ),
        compiler_params=cp,
    )


def _enc_body(x_ref, w_ref, b_ref, o_ref):
    o_ref[...] = x_ref[...] * w_ref[...] + b_ref[...]


def _bn(z, g, b):
    m = jnp.mean(z, axis=0, keepdims=True)
    v = jnp.mean((z - m) ** 2, axis=0, keepdims=True)
    return g * (z - m) / jnp.sqrt(v + 1e-5) + b


def _layer_body(h_ref, a0_ref, a1_ref, ep_ref, w1_ref, b1_ref, g1_ref, be1_ref,
                w2_ref, b2_ref, go_ref, bo_ref, o_ref):
    agg = jnp.concatenate([a0_ref[...], a1_ref[...]], axis=1)
    z = ep_ref[...] * h_ref[...] + agg
    z = jnp.dot(z, w1_ref[...], precision=_DEF,
                preferred_element_type=jnp.float32) + b1_ref[...]
    z = jnp.maximum(_bn(z, g1_ref[...], be1_ref[...]), 0.0)
    z = jnp.dot(z, w2_ref[...], precision=_DEF,
                preferred_element_type=jnp.float32) + b2_ref[...]
    o_ref[...] = jnp.maximum(_bn(z, go_ref[...], bo_ref[...]), 0.0)


def _final_body(h_ref, bt_ref, cw_ref, cb_ref, o_ref):
    ids = lax.broadcasted_iota(jnp.int32, (_G, 1), 0)
    oh = (ids == bt_ref[...]).astype(jnp.float32)
    sums = jnp.dot(oh, h_ref[...], precision=_HIGH,
                   preferred_element_type=jnp.float32)
    cnts = jnp.sum(oh, axis=1, keepdims=True)
    pooled = sums / jnp.maximum(cnts, 1.0)
    o_ref[...] = jnp.dot(pooled, cw_ref[...], precision=_DEF,
                         preferred_element_type=jnp.float32) + cb_ref[...]


_enc_call = pl.pallas_call(
    _enc_body, out_shape=jax.ShapeDtypeStruct((_N, _D), jnp.float32))

_layer_call = pl.pallas_call(
    _layer_body, out_shape=jax.ShapeDtypeStruct((_N, _D), jnp.float32),
    compiler_params=pltpu.CompilerParams(vmem_limit_bytes=64 * 1024 * 1024))

_final_call = pl.pallas_call(
    _final_body, out_shape=jax.ShapeDtypeStruct((_G, 2), jnp.float32))


def kernel(x, edge_index, batch, enc_W, enc_b, W1, b1, g1, be1, W2, b2, eps,
           g_out, b_out, cls_W, cls_b):
    src = edge_index[0]
    dst = edge_index[1]

    b = dst // _ROWS_PT
    oh = (b[:, None] == jnp.arange(_NS, dtype=jnp.int32)[None, :]).astype(jnp.int32)
    csum = jnp.cumsum(oh, axis=0)
    counts = csum[-1]
    offs = jnp.concatenate([jnp.zeros((1,), jnp.int32),
                            jnp.cumsum(counts)[:-1].astype(jnp.int32)])
    newpos = offs[b] + jnp.take_along_axis(csum, b[:, None], axis=1)[:, 0] - 1
    inv = jnp.zeros((_E,), jnp.int32).at[newpos].add(jnp.arange(_E, dtype=jnp.int32))
    srcs = src[inv]
    dsts = dst[inv]
    b0s = offs
    b1s = offs + counts.astype(jnp.int32)
    pad_src = (jnp.arange(_PAD, dtype=jnp.int32) * 97) % _N
    srcs = jnp.concatenate([srcs, pad_src])
    dsts = jnp.concatenate([dsts, jnp.zeros((_PAD,), jnp.int32)])

    h = _enc_call(x, enc_W, enc_b.reshape(1, _D))
    for l in range(_L):
        aggp = _get_sc_agg()(h.reshape(2 * _N, _HALF), srcs, dsts, b0s, b1s)
        h = _layer_call(
            h, aggp[0], aggp[1],
            (1.0 + eps[l]).reshape(1, 1),
            W1[l], b1[l].reshape(1, _D), g1[l].reshape(1, _D),
            be1[l].reshape(1, _D),
            W2[l], b2[l].reshape(1, _D),
            g_out[l].reshape(1, _D), b_out[l].reshape(1, _D),
        )
    return _final_call(h, batch.reshape(1, _N), cls_W, cls_b.reshape(1, 2))
